# baseline (speedup 1.0000x reference)
import functools

import jax
import jax.numpy as jnp
from jax import lax
from jax.experimental import pallas as pl
from jax.experimental.pallas import tpu as pltpu
from jax.experimental.pallas import tpu_sc as plsc

_N = 10000
_E = 320000
_D = 128
_STEPS = 4
_EB = 1000
_NB = 1000
_NTILES = 32
_EPT = _E // _NTILES
_NPAD = 10240
_NPT = _NPAD // 16


def _swish(x):
    return x * jax.nn.sigmoid(x)


def _norm_apply(q, sc, sh):
    mu = jnp.mean(q, axis=-1, keepdims=True)
    qc = q - mu
    var = jnp.mean(qc * qc, axis=-1, keepdims=True)
    return qc * lax.rsqrt(var + 1e-5) * sc + sh


def _full(shape):
    return pl.BlockSpec(shape, lambda i: tuple(0 for _ in shape))


def _rows(nb, d):
    return pl.BlockSpec((nb, d), lambda i: (i, 0))


def _embed_body(x_ref, w1_ref, b1_ref, w2_ref, b2_ref, sc_ref, sh_ref, o_ref):
    h = jnp.dot(x_ref[...], w1_ref[...], preferred_element_type=jnp.float32)
    h = h + b1_ref[...]
    h = _swish(h)
    q = jnp.dot(h, w2_ref[...], preferred_element_type=jnp.float32)
    q = q + b2_ref[...]
    o_ref[...] = _norm_apply(q, sc_ref[...], sh_ref[...])


def _embed_call(x, w1, b1, w2, b2, sc, sh):
    de = x.shape[-1]
    ne = x.shape[0]
    return pl.pallas_call(
        _embed_body,
        grid=(ne // _EB,),
        in_specs=[_rows(_EB, de), _full((de, _D)), _full((1, _D)),
                  _full((_D, _D)), _full((1, _D)), _full((1, _D)),
                  _full((1, _D))],
        out_specs=_rows(_EB, _D),
        out_shape=jax.ShapeDtypeStruct((ne, _D), jnp.float32),
    )(x, w1, b1, w2, b2, sc, sh)


def _proj_body(v_ref, ws_ref, wr_ref, vs_ref, vr_ref):
    v = v_ref[...]
    vs_ref[...] = jnp.dot(v, ws_ref[...], preferred_element_type=jnp.float32)
    vr_ref[...] = jnp.dot(v, wr_ref[...], preferred_element_type=jnp.float32)


def _proj_call(v, ws, wr):
    out = jax.ShapeDtypeStruct((_N, _D), jnp.float32)
    return pl.pallas_call(
        _proj_body,
        grid=(_N // _NB,),
        in_specs=[_rows(_NB, _D), _full((_D, _D)), _full((_D, _D))],
        out_specs=[_rows(_NB, _D), _rows(_NB, _D)],
        out_shape=[out, out],
    )(v, ws, wr)


def _edge_body(e_ref, g_ref, w1_ref, b1_ref, w2_ref, b2_ref,
               sc_ref, sh_ref, o_ref):
    e = e_ref[...]
    h = jnp.dot(e, w1_ref[...], preferred_element_type=jnp.float32)
    h = h + g_ref[...] + b1_ref[...]
    h = _swish(h)
    q = jnp.dot(h, w2_ref[...], preferred_element_type=jnp.float32)
    q = q + b2_ref[...]
    o_ref[...] = e + _norm_apply(q, sc_ref[...], sh_ref[...])


def _edge_call(e, gsum, w1, b1, w2, b2, sc, sh):
    eb = _rows(_EB, _D)
    ne = e.shape[0]
    return pl.pallas_call(
        _edge_body,
        grid=(ne // _EB,),
        in_specs=[eb, eb, _full((_D, _D)), _full((1, _D)),
                  _full((_D, _D)), _full((1, _D)), _full((1, _D)),
                  _full((1, _D))],
        out_specs=eb,
        out_shape=jax.ShapeDtypeStruct((ne, _D), jnp.float32),
    )(e, gsum, w1, b1, w2, b2, sc, sh)


def _node_body(v_ref, p0_ref, p1_ref, p2_ref, p3_ref, d_ref, w1v_ref,
               w1a_ref, b1_ref, w2_ref, b2_ref, sc_ref, sh_ref,
               ws_ref, wr_ref, o_ref, vs_ref, vr_ref):
    v = v_ref[...]
    agg = ((p0_ref[...] + p1_ref[...]) +
           (p2_ref[...] + p3_ref[...])) * d_ref[...]
    h = jnp.dot(v, w1v_ref[...], preferred_element_type=jnp.float32)
    h = h + jnp.dot(agg, w1a_ref[...], preferred_element_type=jnp.float32)
    h = h + b1_ref[...]
    h = _swish(h)
    q = jnp.dot(h, w2_ref[...], preferred_element_type=jnp.float32)
    q = q + b2_ref[...]
    vn = v + _norm_apply(q, sc_ref[...], sh_ref[...])
    o_ref[...] = vn
    vs_ref[...] = jnp.dot(vn, ws_ref[...], preferred_element_type=jnp.float32)
    vr_ref[...] = jnp.dot(vn, wr_ref[...], preferred_element_type=jnp.float32)


def _node_call(v, parts, inv_denom, w1v, w1a, b1, w2, b2, sc, sh, ws, wr):
    nb = _rows(_NB, _D)
    p0, p1, p2, p3 = parts
    out = jax.ShapeDtypeStruct((_N, _D), jnp.float32)
    return pl.pallas_call(
        _node_body,
        grid=(_N // _NB,),
        in_specs=[nb, nb, nb, nb, nb,
                  pl.BlockSpec((_NB, 1), lambda i: (i, 0)),
                  _full((_D, _D)), _full((_D, _D)), _full((1, _D)),
                  _full((_D, _D)), _full((1, _D)), _full((1, _D)),
                  _full((1, _D)), _full((_D, _D)), _full((_D, _D))],
        out_specs=[nb, nb, nb],
        out_shape=[out, out, out],
    )(v, p0, p1, p2, p3, inv_denom, w1v, w1a, b1, w2, b2, sc, sh, ws, wr)


def _sc_mesh():
    return plsc.VectorSubcoreMesh(core_axis_name="c", subcore_axis_name="s")


_GCH = 128
_NFULL = _EPT // _GCH
_TAIL = _EPT - _NFULL * _GCH


def _tile_plan(ept, slots=2, max_gch=128):
    for gch in range(max_gch, 0, -8):
        nfull = ept // gch
        tail = ept - nfull * gch
        if (nfull >= slots and nfull % slots == 0 and tail > 0
                and tail % 8 == 0):
            return gch, nfull, tail
    raise ValueError(ept)


def _sc_gather(vs, vr, senders, receivers, eh=_E):
    ept = eh // _NTILES
    nsl = 3
    gch, nfull, tail = _tile_plan(ept, nsl, max_gch=96)
    buf = pltpu.VMEM((gch, _D), jnp.float32)

    @functools.partial(
        pl.kernel,
        out_type=jax.ShapeDtypeStruct((eh, _D), jnp.float32),
        mesh=_sc_mesh(),
        scratch_types=(
            [pltpu.VMEM((ept,), jnp.int32)] * 2
            + [buf] * (3 * nsl)
            + [pltpu.VMEM((tail, _D), jnp.float32)] * 2
            + [pltpu.SemaphoreType.DMA] * (2 * nsl)
        ),
    )
    def k(vs_hbm, vr_hbm, s_hbm, r_hbm, o_hbm, *refs):
        idx_s, idx_r = refs[0], refs[1]
        bs = refs[2:2 + nsl]
        br = refs[2 + nsl:2 + 2 * nsl]
        wb = refs[2 + 2 * nsl:2 + 3 * nsl]
        tbs, tbr = refs[2 + 3 * nsl], refs[3 + 3 * nsl]
        gsem = refs[4 + 3 * nsl:4 + 4 * nsl]
        wsem = refs[4 + 4 * nsl:4 + 5 * nsl]
        wid = lax.axis_index("s") * 2 + lax.axis_index("c")
        base = wid * ept
        pltpu.sync_copy(s_hbm.at[pl.ds(base, ept)], idx_s)
        pltpu.sync_copy(r_hbm.at[pl.ds(base, ept)], idx_r)

        def gdescs(c, p):
            ds = pltpu.make_async_copy(
                vs_hbm.at[idx_s.at[pl.ds(c * gch, gch)]], bs[p], gsem[p])
            dr = pltpu.make_async_copy(
                vr_hbm.at[idx_r.at[pl.ds(c * gch, gch)]], br[p], gsem[p])
            return ds, dr

        def wdesc(c, p):
            return pltpu.make_async_copy(
                wb[p], o_hbm.at[pl.ds(base + c * gch, gch)], wsem[p])

        for p in range(nsl):
            ds, dr = gdescs(p, p)
            ds.start()
            dr.start()

        @pl.loop(0, nfull, step=nsl)
        def _(i):
            for p in range(nsl):
                c = i + p
                ds, dr = gdescs(c, p)
                ds.wait()
                dr.wait()

                @pl.when(c >= nsl)
                def _():
                    wdesc(c - nsl, p).wait()

                @pl.loop(0, gch)
                def _(r):
                    for j in range(_D // 16):
                        sl = pl.ds(j * 16, 16)
                        wb[p][r, sl] = bs[p][r, sl] + br[p][r, sl]

                wdesc(c, p).start()

                @pl.when(c + nsl < nfull)
                def _():
                    d2, r2 = gdescs(c + nsl, p)
                    d2.start()
                    r2.start()

        for p in range(nsl):
            wdesc(nfull - nsl + p, p).wait()

        tb = nfull * gch
        pltpu.sync_copy(vs_hbm.at[idx_s.at[pl.ds(tb, tail)]], tbs)
        pltpu.sync_copy(vr_hbm.at[idx_r.at[pl.ds(tb, tail)]], tbr)

        @pl.loop(0, tail)
        def _(r):
            for j in range(_D // 16):
                sl = pl.ds(j * 16, 16)
                tbs[r, sl] = tbs[r, sl] + tbr[r, sl]

        pltpu.sync_copy(tbs, o_hbm.at[pl.ds(base + tb, tail)])

    return k(vs, vr, senders, receivers)


def _sc_scatter2(e, receivers, zeros, eh=_E):
    ept = eh // _NTILES
    gch, nfull, tail = _tile_plan(ept)

    @functools.partial(
        pl.kernel,
        out_type=jax.ShapeDtypeStruct((2, _NPAD, _D), jnp.float32),
        mesh=_sc_mesh(),
        scratch_types=[
            pltpu.VMEM((gch,), jnp.int32),
            pltpu.VMEM((gch,), jnp.int32),
            pltpu.VMEM((gch, _D), jnp.float32),
            pltpu.VMEM((gch, _D), jnp.float32),
            pltpu.VMEM((tail,), jnp.int32),
            pltpu.VMEM((tail, _D), jnp.float32),
            pltpu.VMEM_SHARED((_NPAD, _D), jnp.float32),
            pltpu.SemaphoreType.DMA,
            pltpu.SemaphoreType.DMA,
        ],
    )
    def k(e_hbm, r_hbm, z_hbm, o_hbm, ib0, ib1, eb0, eb1, tib, teb, table,
          l0, l1):
        cid = lax.axis_index("c")
        sid = lax.axis_index("s")
        pltpu.sync_copy(z_hbm.at[pl.ds(sid * _NPT, _NPT)],
                        table.at[pl.ds(sid * _NPT, _NPT)])
        plsc.subcore_barrier()
        base = cid * (eh // 2) + sid * ept
        ib = (ib0, ib1)
        eb = (eb0, eb1)
        lsem = (l0, l1)

        def ldescs(c, p):
            b = base + c * gch
            di = pltpu.make_async_copy(r_hbm.at[pl.ds(b, gch)], ib[p],
                                       lsem[p])
            de = pltpu.make_async_copy(e_hbm.at[pl.ds(b, gch)], eb[p],
                                       lsem[p])
            return di, de

        for p in (0, 1):
            di, de = ldescs(p, p)
            di.start()
            de.start()

        @pl.loop(0, nfull, step=2)
        def _(i):
            for p in (0, 1):
                c = i + p
                di, de = ldescs(c, p)
                di.wait()
                de.wait()
                pltpu.sync_copy(eb[p], table.at[ib[p]], add=True)

                @pl.when(c + 2 < nfull)
                def _():
                    d2, e2 = ldescs(c + 2, p)
                    d2.start()
                    e2.start()

        tb = base + nfull * gch
        pltpu.sync_copy(r_hbm.at[pl.ds(tb, tail)], tib)
        pltpu.sync_copy(e_hbm.at[pl.ds(tb, tail)], teb)
        pltpu.sync_copy(teb, table.at[tib], add=True)

        plsc.subcore_barrier()
        pltpu.sync_copy(table.at[pl.ds(sid * _NPT, _NPT)],
                        o_hbm.at[cid, pl.ds(sid * _NPT, _NPT)])

    return k(e, receivers, zeros)


def _sc_counts(receivers, ones_chunk, zeros16):
    @functools.partial(
        pl.kernel,
        out_type=jax.ShapeDtypeStruct((2, _NPAD, _D), jnp.float32),
        mesh=_sc_mesh(),
        scratch_types=[
            pltpu.VMEM((_GCH,), jnp.int32),
            pltpu.VMEM((_GCH,), jnp.int32),
            pltpu.VMEM((_TAIL,), jnp.int32),
            pltpu.VMEM((_GCH, _D), jnp.float32),
            pltpu.VMEM_SHARED((_NPAD, _D), jnp.float32),
            pltpu.SemaphoreType.DMA,
            pltpu.SemaphoreType.DMA,
        ],
    )
    def k(r_hbm, ones_hbm, z_hbm, o_hbm, ib0, ib1, tib, ones_v, table,
          l0, l1):
        cid = lax.axis_index("c")
        sid = lax.axis_index("s")
        pltpu.sync_copy(ones_hbm, ones_v)
        pltpu.sync_copy(z_hbm.at[pl.ds(sid * _NPT, _NPT)],
                        table.at[pl.ds(sid * _NPT, _NPT)])
        plsc.subcore_barrier()
        base = cid * (_E // 2) + sid * _EPT
        ib = (ib0, ib1)
        lsem = (l0, l1)

        def idesc(c, p):
            return pltpu.make_async_copy(
                r_hbm.at[pl.ds(base + c * _GCH, _GCH)], ib[p], lsem[p])

        for p in (0, 1):
            idesc(p, p).start()

        @pl.loop(0, _NFULL, step=2)
        def _(i):
            for p in (0, 1):
                c = i + p
                idesc(c, p).wait()
                pltpu.sync_copy(ones_v, table.at[ib[p]], add=True)

                @pl.when(c + 2 < _NFULL)
                def _():
                    idesc(c + 2, p).start()

        tb = base + _NFULL * _GCH
        pltpu.sync_copy(r_hbm.at[pl.ds(tb, _TAIL)], tib)
        pltpu.sync_copy(ones_v.at[pl.ds(0, _TAIL)], table.at[tib], add=True)

        plsc.subcore_barrier()
        pltpu.sync_copy(table.at[pl.ds(sid * _NPT, _NPT)],
                        o_hbm.at[cid, pl.ds(sid * _NPT, _NPT)])

    return k(receivers, ones_chunk, zeros16)


def kernel(rnode_features, edge_features, senders, receivers, tau,
           emb_W1, emb_b1, emb_W2, emb_b2, emb_cW1, emb_cb1, emb_cW2,
           emb_cb2, e_W1, e_b1, e_W2, e_b2, e_cW1, e_cb1, e_cW2, e_cb2,
           n_W1, n_b1, n_W2, n_b2, n_cW1, n_cb1, n_cW2, n_cb2):
    v = rnode_features[0]
    ef = edge_features[0]

    def cond(cW1, cb1, cW2, cb2):
        p = _swish(tau @ cW1 + cb1) @ cW2 + cb2
        return 1.0 + p[:, :_D], p[:, _D:]

    zeros = jnp.zeros((_NPAD, _D), jnp.float32)
    ones_chunk = jnp.ones((_GCH, _D), jnp.float32)

    counts2 = _sc_counts(receivers, ones_chunk, zeros)
    counts = counts2[0, :, 0:1] + counts2[1, :, 0:1]
    inv_denom = 1.0 / jnp.maximum(counts, 1.0)

    _H = _E // 2
    s_halves = (senders[:_H], senders[_H:])
    r_halves = (receivers[:_H], receivers[_H:])

    emb_sc, emb_sh = cond(emb_cW1, emb_cb1, emb_cW2, emb_cb2)
    e_h = [
        _embed_call(ef[:_H], emb_W1, emb_b1[None], emb_W2, emb_b2[None],
                    emb_sc, emb_sh),
        _embed_call(ef[_H:], emb_W1, emb_b1[None], emb_W2, emb_b2[None],
                    emb_sc, emb_sh),
    ]

    w1splits = [(e_W1[st][:_D], e_W1[st][_D:2 * _D], e_W1[st][2 * _D:])
                for st in range(_STEPS)]
    vs, vr = _proj_call(v, w1splits[0][1], w1splits[0][2])
    for st in range(_STEPS):
        w1e = w1splits[st][0]
        e_sc, e_sh = cond(e_cW1[st], e_cb1[st], e_cW2[st], e_cb2[st])
        gsums = [_sc_gather(vs, vr, s_halves[h], r_halves[h], _H)
                 for h in range(2)]
        parts = []
        for h in range(2):
            e_h[h] = _edge_call(e_h[h], gsums[h], w1e, e_b1[st][None],
                                e_W2[st], e_b2[st][None], e_sc, e_sh)
            parts.append(_sc_scatter2(e_h[h], r_halves[h], zeros, _H))
        n1 = n_W1[st]
        w1v, w1a = n1[:_D], n1[_D:]
        n_sc, n_sh = cond(n_cW1[st], n_cb1[st], n_cW2[st], n_cb2[st])
        nst = (st + 1) % _STEPS
        v, vs, vr = _node_call(
            v, (parts[0][0], parts[0][1], parts[1][0], parts[1][1]),
            inv_denom, w1v, w1a, n_b1[st][None], n_W2[st], n_b2[st][None],
            n_sc, n_sh, w1splits[nst][1], w1splits[nst][2])

    return v[None]


# speedup vs baseline: 1.2216x; 1.2216x over previous
import functools

import jax
import jax.numpy as jnp
from jax import lax
from jax.experimental import pallas as pl
from jax.experimental.pallas import tpu as pltpu
from jax.experimental.pallas import tpu_sc as plsc

_N = 10000
_E = 320000
_D = 128
_STEPS = 4
_EB = 2000
_NB = 1000
_NTILES = 32
_EPT = _E // _NTILES
_NPAD = 10240
_NPT = _NPAD // 16


def _swish(x):
    return x * jax.nn.sigmoid(x)


def _norm_apply(q, sc, sh):
    mu = jnp.mean(q, axis=-1, keepdims=True)
    qc = q - mu
    var = jnp.mean(qc * qc, axis=-1, keepdims=True)
    return qc * lax.rsqrt(var + 1e-5) * sc + sh


def _full(shape):
    return pl.BlockSpec(shape, lambda i: tuple(0 for _ in shape))


def _rows(nb, d):
    return pl.BlockSpec((nb, d), lambda i: (i, 0))


def _embed_body(x_ref, w1_ref, b1_ref, w2_ref, b2_ref, sc_ref, sh_ref, o_ref):
    h = jnp.dot(x_ref[...], w1_ref[...], preferred_element_type=jnp.float32)
    h = h + b1_ref[...]
    h = _swish(h)
    q = jnp.dot(h, w2_ref[...], preferred_element_type=jnp.float32)
    q = q + b2_ref[...]
    o_ref[...] = _norm_apply(q, sc_ref[...], sh_ref[...])


def _embed_call(x, w1, b1, w2, b2, sc, sh):
    de = x.shape[-1]
    ne = x.shape[0]
    return pl.pallas_call(
        _embed_body,
        grid=(ne // _EB,),
        in_specs=[_rows(_EB, de), _full((de, _D)), _full((1, _D)),
                  _full((_D, _D)), _full((1, _D)), _full((1, _D)),
                  _full((1, _D))],
        out_specs=_rows(_EB, _D),
        out_shape=jax.ShapeDtypeStruct((ne, _D), jnp.float32),
    )(x, w1, b1, w2, b2, sc, sh)


def _proj_body(v_ref, ws_ref, wr_ref, vs_ref, vr_ref):
    v = v_ref[...]
    vs_ref[...] = jnp.dot(v, ws_ref[...], preferred_element_type=jnp.float32)
    vr_ref[...] = jnp.dot(v, wr_ref[...], preferred_element_type=jnp.float32)


def _proj_call(v, ws, wr):
    out = jax.ShapeDtypeStruct((_N, _D), jnp.float32)
    return pl.pallas_call(
        _proj_body,
        grid=(_N // _NB,),
        in_specs=[_rows(_NB, _D), _full((_D, _D)), _full((_D, _D))],
        out_specs=[_rows(_NB, _D), _rows(_NB, _D)],
        out_shape=[out, out],
    )(v, ws, wr)


def _edge_body(e_ref, g_ref, w1_ref, b1_ref, w2_ref, b2_ref,
               sc_ref, sh_ref, o_ref):
    e = e_ref[...]
    h = jnp.dot(e, w1_ref[...], preferred_element_type=jnp.float32)
    h = h + g_ref[...] + b1_ref[...]
    h = _swish(h)
    q = jnp.dot(h, w2_ref[...], preferred_element_type=jnp.float32)
    q = q + b2_ref[...]
    o_ref[...] = e + _norm_apply(q, sc_ref[...], sh_ref[...])


def _edge_call(e, gsum, w1, b1, w2, b2, sc, sh):
    eb = _rows(_EB, _D)
    ne = e.shape[0]
    return pl.pallas_call(
        _edge_body,
        grid=(ne // _EB,),
        in_specs=[eb, eb, _full((_D, _D)), _full((1, _D)),
                  _full((_D, _D)), _full((1, _D)), _full((1, _D)),
                  _full((1, _D))],
        out_specs=eb,
        out_shape=jax.ShapeDtypeStruct((ne, _D), jnp.float32),
    )(e, gsum, w1, b1, w2, b2, sc, sh)


def _node_body(v_ref, p0_ref, p1_ref, p2_ref, p3_ref, d_ref, w1v_ref,
               w1a_ref, b1_ref, w2_ref, b2_ref, sc_ref, sh_ref,
               ws_ref, wr_ref, o_ref, vs_ref, vr_ref):
    v = v_ref[...]
    agg = ((p0_ref[...] + p1_ref[...]) +
           (p2_ref[...] + p3_ref[...])) * d_ref[...]
    h = jnp.dot(v, w1v_ref[...], preferred_element_type=jnp.float32)
    h = h + jnp.dot(agg, w1a_ref[...], preferred_element_type=jnp.float32)
    h = h + b1_ref[...]
    h = _swish(h)
    q = jnp.dot(h, w2_ref[...], preferred_element_type=jnp.float32)
    q = q + b2_ref[...]
    vn = v + _norm_apply(q, sc_ref[...], sh_ref[...])
    o_ref[...] = vn
    vs_ref[...] = jnp.dot(vn, ws_ref[...], preferred_element_type=jnp.float32)
    vr_ref[...] = jnp.dot(vn, wr_ref[...], preferred_element_type=jnp.float32)


def _node_call(v, parts, inv_denom, w1v, w1a, b1, w2, b2, sc, sh, ws, wr):
    nb = _rows(_NB, _D)
    p0, p1, p2, p3 = parts
    out = jax.ShapeDtypeStruct((_N, _D), jnp.float32)
    return pl.pallas_call(
        _node_body,
        grid=(_N // _NB,),
        in_specs=[nb, nb, nb, nb, nb,
                  pl.BlockSpec((_NB, 1), lambda i: (i, 0)),
                  _full((_D, _D)), _full((_D, _D)), _full((1, _D)),
                  _full((_D, _D)), _full((1, _D)), _full((1, _D)),
                  _full((1, _D)), _full((_D, _D)), _full((_D, _D))],
        out_specs=[nb, nb, nb],
        out_shape=[out, out, out],
    )(v, p0, p1, p2, p3, inv_denom, w1v, w1a, b1, w2, b2, sc, sh, ws, wr)


def _sc_mesh():
    return plsc.VectorSubcoreMesh(core_axis_name="c", subcore_axis_name="s")


_GCH = 128
_NFULL = _EPT // _GCH
_TAIL = _EPT - _NFULL * _GCH


def _tile_plan(ept, slots=2, max_gch=128):
    for gch in range(max_gch, 0, -8):
        nfull = ept // gch
        tail = ept - nfull * gch
        if (nfull >= slots and nfull % slots == 0 and tail > 0
                and tail % 8 == 0):
            return gch, nfull, tail
    raise ValueError(ept)


def _sc_gather(vs, vr, senders, receivers, eh=_E):
    ept = eh // _NTILES
    nsl = 3
    gch, nfull, tail = _tile_plan(ept, nsl, max_gch=96)
    buf = pltpu.VMEM((gch, _D), jnp.float32)

    @functools.partial(
        pl.kernel,
        out_type=jax.ShapeDtypeStruct((eh, _D), jnp.float32),
        mesh=_sc_mesh(),
        scratch_types=(
            [pltpu.VMEM((ept,), jnp.int32)] * 2
            + [buf] * (3 * nsl)
            + [pltpu.VMEM((tail, _D), jnp.float32)] * 2
            + [pltpu.SemaphoreType.DMA] * (2 * nsl)
        ),
    )
    def k(vs_hbm, vr_hbm, s_hbm, r_hbm, o_hbm, *refs):
        idx_s, idx_r = refs[0], refs[1]
        bs = refs[2:2 + nsl]
        br = refs[2 + nsl:2 + 2 * nsl]
        wb = refs[2 + 2 * nsl:2 + 3 * nsl]
        tbs, tbr = refs[2 + 3 * nsl], refs[3 + 3 * nsl]
        gsem = refs[4 + 3 * nsl:4 + 4 * nsl]
        wsem = refs[4 + 4 * nsl:4 + 5 * nsl]
        wid = lax.axis_index("s") * 2 + lax.axis_index("c")
        base = wid * ept
        pltpu.sync_copy(s_hbm.at[pl.ds(base, ept)], idx_s)
        pltpu.sync_copy(r_hbm.at[pl.ds(base, ept)], idx_r)

        def gdescs(c, p):
            ds = pltpu.make_async_copy(
                vs_hbm.at[idx_s.at[pl.ds(c * gch, gch)]], bs[p], gsem[p])
            dr = pltpu.make_async_copy(
                vr_hbm.at[idx_r.at[pl.ds(c * gch, gch)]], br[p], gsem[p])
            return ds, dr

        def wdesc(c, p):
            return pltpu.make_async_copy(
                wb[p], o_hbm.at[pl.ds(base + c * gch, gch)], wsem[p])

        for p in range(nsl):
            ds, dr = gdescs(p, p)
            ds.start()
            dr.start()

        @pl.loop(0, nfull, step=nsl)
        def _(i):
            for p in range(nsl):
                c = i + p
                ds, dr = gdescs(c, p)
                ds.wait()
                dr.wait()

                @pl.when(c >= nsl)
                def _():
                    wdesc(c - nsl, p).wait()

                @pl.loop(0, gch)
                def _(r):
                    for j in range(_D // 16):
                        sl = pl.ds(j * 16, 16)
                        wb[p][r, sl] = bs[p][r, sl] + br[p][r, sl]

                wdesc(c, p).start()

                @pl.when(c + nsl < nfull)
                def _():
                    d2, r2 = gdescs(c + nsl, p)
                    d2.start()
                    r2.start()

        for p in range(nsl):
            wdesc(nfull - nsl + p, p).wait()

        tb = nfull * gch
        pltpu.sync_copy(vs_hbm.at[idx_s.at[pl.ds(tb, tail)]], tbs)
        pltpu.sync_copy(vr_hbm.at[idx_r.at[pl.ds(tb, tail)]], tbr)

        @pl.loop(0, tail)
        def _(r):
            for j in range(_D // 16):
                sl = pl.ds(j * 16, 16)
                tbs[r, sl] = tbs[r, sl] + tbr[r, sl]

        pltpu.sync_copy(tbs, o_hbm.at[pl.ds(base + tb, tail)])

    return k(vs, vr, senders, receivers)


def _sc_scatter2(e, receivers, zeros, eh=_E):
    ept = eh // _NTILES
    gch, nfull, tail = _tile_plan(ept)

    @functools.partial(
        pl.kernel,
        out_type=jax.ShapeDtypeStruct((2, _NPAD, _D), jnp.float32),
        mesh=_sc_mesh(),
        scratch_types=[
            pltpu.VMEM((gch,), jnp.int32),
            pltpu.VMEM((gch,), jnp.int32),
            pltpu.VMEM((gch, _D), jnp.float32),
            pltpu.VMEM((gch, _D), jnp.float32),
            pltpu.VMEM((tail,), jnp.int32),
            pltpu.VMEM((tail, _D), jnp.float32),
            pltpu.VMEM_SHARED((_NPAD, _D), jnp.float32),
            pltpu.SemaphoreType.DMA,
            pltpu.SemaphoreType.DMA,
        ],
    )
    def k(e_hbm, r_hbm, z_hbm, o_hbm, ib0, ib1, eb0, eb1, tib, teb, table,
          l0, l1):
        cid = lax.axis_index("c")
        sid = lax.axis_index("s")
        pltpu.sync_copy(z_hbm.at[pl.ds(sid * _NPT, _NPT)],
                        table.at[pl.ds(sid * _NPT, _NPT)])
        plsc.subcore_barrier()
        base = cid * (eh // 2) + sid * ept
        ib = (ib0, ib1)
        eb = (eb0, eb1)
        lsem = (l0, l1)

        def ldescs(c, p):
            b = base + c * gch
            di = pltpu.make_async_copy(r_hbm.at[pl.ds(b, gch)], ib[p],
                                       lsem[p])
            de = pltpu.make_async_copy(e_hbm.at[pl.ds(b, gch)], eb[p],
                                       lsem[p])
            return di, de

        for p in (0, 1):
            di, de = ldescs(p, p)
            di.start()
            de.start()

        @pl.loop(0, nfull, step=2)
        def _(i):
            for p in (0, 1):
                c = i + p
                di, de = ldescs(c, p)
                di.wait()
                de.wait()
                pltpu.sync_copy(eb[p], table.at[ib[p]], add=True)

                @pl.when(c + 2 < nfull)
                def _():
                    d2, e2 = ldescs(c + 2, p)
                    d2.start()
                    e2.start()

        tb = base + nfull * gch
        pltpu.sync_copy(r_hbm.at[pl.ds(tb, tail)], tib)
        pltpu.sync_copy(e_hbm.at[pl.ds(tb, tail)], teb)
        pltpu.sync_copy(teb, table.at[tib], add=True)

        plsc.subcore_barrier()
        pltpu.sync_copy(table.at[pl.ds(sid * _NPT, _NPT)],
                        o_hbm.at[cid, pl.ds(sid * _NPT, _NPT)])

    return k(e, receivers, zeros)


def _sc_counts(receivers, ones_chunk, zeros16):
    @functools.partial(
        pl.kernel,
        out_type=jax.ShapeDtypeStruct((2, _NPAD, _D), jnp.float32),
        mesh=_sc_mesh(),
        scratch_types=[
            pltpu.VMEM((_GCH,), jnp.int32),
            pltpu.VMEM((_GCH,), jnp.int32),
            pltpu.VMEM((_TAIL,), jnp.int32),
            pltpu.VMEM((_GCH, _D), jnp.float32),
            pltpu.VMEM_SHARED((_NPAD, _D), jnp.float32),
            pltpu.SemaphoreType.DMA,
            pltpu.SemaphoreType.DMA,
        ],
    )
    def k(r_hbm, ones_hbm, z_hbm, o_hbm, ib0, ib1, tib, ones_v, table,
          l0, l1):
        cid = lax.axis_index("c")
        sid = lax.axis_index("s")
        pltpu.sync_copy(ones_hbm, ones_v)
        pltpu.sync_copy(z_hbm.at[pl.ds(sid * _NPT, _NPT)],
                        table.at[pl.ds(sid * _NPT, _NPT)])
        plsc.subcore_barrier()
        base = cid * (_E // 2) + sid * _EPT
        ib = (ib0, ib1)
        lsem = (l0, l1)

        def idesc(c, p):
            return pltpu.make_async_copy(
                r_hbm.at[pl.ds(base + c * _GCH, _GCH)], ib[p], lsem[p])

        for p in (0, 1):
            idesc(p, p).start()

        @pl.loop(0, _NFULL, step=2)
        def _(i):
            for p in (0, 1):
                c = i + p
                idesc(c, p).wait()
                pltpu.sync_copy(ones_v, table.at[ib[p]], add=True)

                @pl.when(c + 2 < _NFULL)
                def _():
                    idesc(c + 2, p).start()

        tb = base + _NFULL * _GCH
        pltpu.sync_copy(r_hbm.at[pl.ds(tb, _TAIL)], tib)
        pltpu.sync_copy(ones_v.at[pl.ds(0, _TAIL)], table.at[tib], add=True)

        plsc.subcore_barrier()
        pltpu.sync_copy(table.at[pl.ds(sid * _NPT, _NPT)],
                        o_hbm.at[cid, pl.ds(sid * _NPT, _NPT)])

    return k(receivers, ones_chunk, zeros16)


def kernel(rnode_features, edge_features, senders, receivers, tau,
           emb_W1, emb_b1, emb_W2, emb_b2, emb_cW1, emb_cb1, emb_cW2,
           emb_cb2, e_W1, e_b1, e_W2, e_b2, e_cW1, e_cb1, e_cW2, e_cb2,
           n_W1, n_b1, n_W2, n_b2, n_cW1, n_cb1, n_cW2, n_cb2):
    v = rnode_features[0]
    ef = edge_features[0]

    def cond(cW1, cb1, cW2, cb2):
        p = _swish(tau @ cW1 + cb1) @ cW2 + cb2
        return 1.0 + p[:, :_D], p[:, _D:]

    zeros = jnp.zeros((_NPAD, _D), jnp.float32)
    ones_chunk = jnp.ones((_GCH, _D), jnp.float32)

    counts2 = _sc_counts(receivers, ones_chunk, zeros)
    counts = counts2[0, :, 0:1] + counts2[1, :, 0:1]
    inv_denom = 1.0 / jnp.maximum(counts, 1.0)

    _H = _E // 2
    s_halves = (senders[:_H], senders[_H:])
    r_halves = (receivers[:_H], receivers[_H:])

    emb_sc, emb_sh = cond(emb_cW1, emb_cb1, emb_cW2, emb_cb2)
    e_h = [
        _embed_call(ef[:_H], emb_W1, emb_b1[None], emb_W2, emb_b2[None],
                    emb_sc, emb_sh),
        _embed_call(ef[_H:], emb_W1, emb_b1[None], emb_W2, emb_b2[None],
                    emb_sc, emb_sh),
    ]

    w1splits = [(e_W1[st][:_D], e_W1[st][_D:2 * _D], e_W1[st][2 * _D:])
                for st in range(_STEPS)]
    vs, vr = _proj_call(v, w1splits[0][1], w1splits[0][2])
    for st in range(_STEPS):
        w1e = w1splits[st][0]
        e_sc, e_sh = cond(e_cW1[st], e_cb1[st], e_cW2[st], e_cb2[st])
        gsums = [_sc_gather(vs, vr, s_halves[h], r_halves[h], _H)
                 for h in range(2)]
        parts = []
        for h in range(2):
            e_h[h] = _edge_call(e_h[h], gsums[h], w1e, e_b1[st][None],
                                e_W2[st], e_b2[st][None], e_sc, e_sh)
            parts.append(_sc_scatter2(e_h[h], r_halves[h], zeros, _H))
        n1 = n_W1[st]
        w1v, w1a = n1[:_D], n1[_D:]
        n_sc, n_sh = cond(n_cW1[st], n_cb1[st], n_cW2[st], n_cb2[st])
        nst = (st + 1) % _STEPS
        v, vs, vr = _node_call(
            v, (parts[0][0], parts[0][1], parts[1][0], parts[1][1]),
            inv_denom, w1v, w1a, n_b1[st][None], n_W2[st], n_b2[st][None],
            n_sc, n_sh, w1splits[nst][1], w1splits[nst][2])

    return v[None]


# speedup vs baseline: 1.3229x; 1.0829x over previous
import functools

import jax
import jax.numpy as jnp
from jax import lax
from jax.experimental import pallas as pl
from jax.experimental.pallas import tpu as pltpu
from jax.experimental.pallas import tpu_sc as plsc

_N = 10000
_E = 320000
_D = 128
_STEPS = 4
_EB = 4000
_NB = 2000
_NTILES = 32
_EPT = _E // _NTILES
_NPAD = 10240
_NPT = _NPAD // 16


def _swish(x):
    return x * jax.nn.sigmoid(x)


def _norm_apply(q, sc, sh):
    mu = jnp.mean(q, axis=-1, keepdims=True)
    qc = q - mu
    var = jnp.mean(qc * qc, axis=-1, keepdims=True)
    return qc * lax.rsqrt(var + 1e-5) * sc + sh


def _full(shape):
    return pl.BlockSpec(shape, lambda i: tuple(0 for _ in shape))


def _rows(nb, d):
    return pl.BlockSpec((nb, d), lambda i: (i, 0))


def _embed_body(x_ref, w1_ref, b1_ref, w2_ref, b2_ref, sc_ref, sh_ref, o_ref):
    h = jnp.dot(x_ref[...], w1_ref[...], preferred_element_type=jnp.float32)
    h = h + b1_ref[...]
    h = _swish(h)
    q = jnp.dot(h, w2_ref[...], preferred_element_type=jnp.float32)
    q = q + b2_ref[...]
    o_ref[...] = _norm_apply(q, sc_ref[...], sh_ref[...])


def _embed_call(x, w1, b1, w2, b2, sc, sh):
    de = x.shape[-1]
    ne = x.shape[0]
    return pl.pallas_call(
        _embed_body,
        grid=(ne // _EB,),
        in_specs=[_rows(_EB, de), _full((de, _D)), _full((1, _D)),
                  _full((_D, _D)), _full((1, _D)), _full((1, _D)),
                  _full((1, _D))],
        out_specs=_rows(_EB, _D),
        out_shape=jax.ShapeDtypeStruct((ne, _D), jnp.float32),
    )(x, w1, b1, w2, b2, sc, sh)


def _proj_body(v_ref, ws_ref, wr_ref, vs_ref, vr_ref):
    v = v_ref[...]
    vs_ref[...] = jnp.dot(v, ws_ref[...], preferred_element_type=jnp.float32)
    vr_ref[...] = jnp.dot(v, wr_ref[...], preferred_element_type=jnp.float32)


def _proj_call(v, ws, wr):
    out = jax.ShapeDtypeStruct((_N, _D), jnp.float32)
    return pl.pallas_call(
        _proj_body,
        grid=(_N // _NB,),
        in_specs=[_rows(_NB, _D), _full((_D, _D)), _full((_D, _D))],
        out_specs=[_rows(_NB, _D), _rows(_NB, _D)],
        out_shape=[out, out],
    )(v, ws, wr)


def _edge_body(e_ref, g_ref, w1_ref, b1_ref, w2_ref, b2_ref,
               sc_ref, sh_ref, o_ref):
    e = e_ref[...]
    h = jnp.dot(e, w1_ref[...], preferred_element_type=jnp.float32)
    h = h + g_ref[...] + b1_ref[...]
    h = _swish(h)
    q = jnp.dot(h, w2_ref[...], preferred_element_type=jnp.float32)
    q = q + b2_ref[...]
    o_ref[...] = e + _norm_apply(q, sc_ref[...], sh_ref[...])


def _edge_call(e, gsum, w1, b1, w2, b2, sc, sh):
    eb = _rows(_EB, _D)
    ne = e.shape[0]
    return pl.pallas_call(
        _edge_body,
        grid=(ne // _EB,),
        in_specs=[eb, eb, _full((_D, _D)), _full((1, _D)),
                  _full((_D, _D)), _full((1, _D)), _full((1, _D)),
                  _full((1, _D))],
        out_specs=eb,
        out_shape=jax.ShapeDtypeStruct((ne, _D), jnp.float32),
    )(e, gsum, w1, b1, w2, b2, sc, sh)


def _node_body(v_ref, p0_ref, p1_ref, p2_ref, p3_ref, d_ref, w1v_ref,
               w1a_ref, b1_ref, w2_ref, b2_ref, sc_ref, sh_ref,
               ws_ref, wr_ref, o_ref, vs_ref, vr_ref):
    v = v_ref[...]
    agg = ((p0_ref[...] + p1_ref[...]) +
           (p2_ref[...] + p3_ref[...])) * d_ref[...]
    h = jnp.dot(v, w1v_ref[...], preferred_element_type=jnp.float32)
    h = h + jnp.dot(agg, w1a_ref[...], preferred_element_type=jnp.float32)
    h = h + b1_ref[...]
    h = _swish(h)
    q = jnp.dot(h, w2_ref[...], preferred_element_type=jnp.float32)
    q = q + b2_ref[...]
    vn = v + _norm_apply(q, sc_ref[...], sh_ref[...])
    o_ref[...] = vn
    vs_ref[...] = jnp.dot(vn, ws_ref[...], preferred_element_type=jnp.float32)
    vr_ref[...] = jnp.dot(vn, wr_ref[...], preferred_element_type=jnp.float32)


def _node_call(v, parts, inv_denom, w1v, w1a, b1, w2, b2, sc, sh, ws, wr):
    nb = _rows(_NB, _D)
    p0, p1, p2, p3 = parts
    out = jax.ShapeDtypeStruct((_N, _D), jnp.float32)
    return pl.pallas_call(
        _node_body,
        grid=(_N // _NB,),
        in_specs=[nb, nb, nb, nb, nb,
                  pl.BlockSpec((_NB, 1), lambda i: (i, 0)),
                  _full((_D, _D)), _full((_D, _D)), _full((1, _D)),
                  _full((_D, _D)), _full((1, _D)), _full((1, _D)),
                  _full((1, _D)), _full((_D, _D)), _full((_D, _D))],
        out_specs=[nb, nb, nb],
        out_shape=[out, out, out],
    )(v, p0, p1, p2, p3, inv_denom, w1v, w1a, b1, w2, b2, sc, sh, ws, wr)


def _sc_mesh():
    return plsc.VectorSubcoreMesh(core_axis_name="c", subcore_axis_name="s")


_GCH = 128
_NFULL = _EPT // _GCH
_TAIL = _EPT - _NFULL * _GCH


def _tile_plan(ept, slots=2, max_gch=128):
    for gch in range(max_gch, 0, -8):
        nfull = ept // gch
        tail = ept - nfull * gch
        if (nfull >= slots and nfull % slots == 0 and tail > 0
                and tail % 8 == 0):
            return gch, nfull, tail
    raise ValueError(ept)


def _sc_gather(vs, vr, senders, receivers, eh=_E):
    ept = eh // _NTILES
    nsl = 3
    gch, nfull, tail = _tile_plan(ept, nsl, max_gch=96)
    buf = pltpu.VMEM((gch, _D), jnp.float32)

    @functools.partial(
        pl.kernel,
        out_type=jax.ShapeDtypeStruct((eh, _D), jnp.float32),
        mesh=_sc_mesh(),
        scratch_types=(
            [pltpu.VMEM((ept,), jnp.int32)] * 2
            + [buf] * (3 * nsl)
            + [pltpu.VMEM((tail, _D), jnp.float32)] * 2
            + [pltpu.SemaphoreType.DMA] * (2 * nsl)
        ),
    )
    def k(vs_hbm, vr_hbm, s_hbm, r_hbm, o_hbm, *refs):
        idx_s, idx_r = refs[0], refs[1]
        bs = refs[2:2 + nsl]
        br = refs[2 + nsl:2 + 2 * nsl]
        wb = refs[2 + 2 * nsl:2 + 3 * nsl]
        tbs, tbr = refs[2 + 3 * nsl], refs[3 + 3 * nsl]
        gsem = refs[4 + 3 * nsl:4 + 4 * nsl]
        wsem = refs[4 + 4 * nsl:4 + 5 * nsl]
        wid = lax.axis_index("s") * 2 + lax.axis_index("c")
        base = wid * ept
        pltpu.sync_copy(s_hbm.at[pl.ds(base, ept)], idx_s)
        pltpu.sync_copy(r_hbm.at[pl.ds(base, ept)], idx_r)

        def gdescs(c, p):
            ds = pltpu.make_async_copy(
                vs_hbm.at[idx_s.at[pl.ds(c * gch, gch)]], bs[p], gsem[p])
            dr = pltpu.make_async_copy(
                vr_hbm.at[idx_r.at[pl.ds(c * gch, gch)]], br[p], gsem[p])
            return ds, dr

        def wdesc(c, p):
            return pltpu.make_async_copy(
                wb[p], o_hbm.at[pl.ds(base + c * gch, gch)], wsem[p])

        for p in range(nsl):
            ds, dr = gdescs(p, p)
            ds.start()
            dr.start()

        @pl.loop(0, nfull, step=nsl)
        def _(i):
            for p in range(nsl):
                c = i + p
                ds, dr = gdescs(c, p)
                ds.wait()
                dr.wait()

                @pl.when(c >= nsl)
                def _():
                    wdesc(c - nsl, p).wait()

                @pl.loop(0, gch)
                def _(r):
                    for j in range(_D // 16):
                        sl = pl.ds(j * 16, 16)
                        wb[p][r, sl] = bs[p][r, sl] + br[p][r, sl]

                wdesc(c, p).start()

                @pl.when(c + nsl < nfull)
                def _():
                    d2, r2 = gdescs(c + nsl, p)
                    d2.start()
                    r2.start()

        for p in range(nsl):
            wdesc(nfull - nsl + p, p).wait()

        tb = nfull * gch
        pltpu.sync_copy(vs_hbm.at[idx_s.at[pl.ds(tb, tail)]], tbs)
        pltpu.sync_copy(vr_hbm.at[idx_r.at[pl.ds(tb, tail)]], tbr)

        @pl.loop(0, tail)
        def _(r):
            for j in range(_D // 16):
                sl = pl.ds(j * 16, 16)
                tbs[r, sl] = tbs[r, sl] + tbr[r, sl]

        pltpu.sync_copy(tbs, o_hbm.at[pl.ds(base + tb, tail)])

    return k(vs, vr, senders, receivers)


def _sc_scatter2(e, receivers, zeros, eh=_E):
    ept = eh // _NTILES
    gch, nfull, tail = _tile_plan(ept)

    @functools.partial(
        pl.kernel,
        out_type=jax.ShapeDtypeStruct((2, _NPAD, _D), jnp.float32),
        mesh=_sc_mesh(),
        scratch_types=[
            pltpu.VMEM((gch,), jnp.int32),
            pltpu.VMEM((gch,), jnp.int32),
            pltpu.VMEM((gch, _D), jnp.float32),
            pltpu.VMEM((gch, _D), jnp.float32),
            pltpu.VMEM((tail,), jnp.int32),
            pltpu.VMEM((tail, _D), jnp.float32),
            pltpu.VMEM_SHARED((_NPAD, _D), jnp.float32),
            pltpu.SemaphoreType.DMA,
            pltpu.SemaphoreType.DMA,
        ],
    )
    def k(e_hbm, r_hbm, z_hbm, o_hbm, ib0, ib1, eb0, eb1, tib, teb, table,
          l0, l1):
        cid = lax.axis_index("c")
        sid = lax.axis_index("s")
        pltpu.sync_copy(z_hbm.at[pl.ds(sid * _NPT, _NPT)],
                        table.at[pl.ds(sid * _NPT, _NPT)])
        plsc.subcore_barrier()
        base = cid * (eh // 2) + sid * ept
        ib = (ib0, ib1)
        eb = (eb0, eb1)
        lsem = (l0, l1)

        def ldescs(c, p):
            b = base + c * gch
            di = pltpu.make_async_copy(r_hbm.at[pl.ds(b, gch)], ib[p],
                                       lsem[p])
            de = pltpu.make_async_copy(e_hbm.at[pl.ds(b, gch)], eb[p],
                                       lsem[p])
            return di, de

        for p in (0, 1):
            di, de = ldescs(p, p)
            di.start()
            de.start()

        @pl.loop(0, nfull, step=2)
        def _(i):
            for p in (0, 1):
                c = i + p
                di, de = ldescs(c, p)
                di.wait()
                de.wait()
                pltpu.sync_copy(eb[p], table.at[ib[p]], add=True)

                @pl.when(c + 2 < nfull)
                def _():
                    d2, e2 = ldescs(c + 2, p)
                    d2.start()
                    e2.start()

        tb = base + nfull * gch
        pltpu.sync_copy(r_hbm.at[pl.ds(tb, tail)], tib)
        pltpu.sync_copy(e_hbm.at[pl.ds(tb, tail)], teb)
        pltpu.sync_copy(teb, table.at[tib], add=True)

        plsc.subcore_barrier()
        pltpu.sync_copy(table.at[pl.ds(sid * _NPT, _NPT)],
                        o_hbm.at[cid, pl.ds(sid * _NPT, _NPT)])

    return k(e, receivers, zeros)


def _sc_counts(receivers, ones_chunk, zeros16):
    @functools.partial(
        pl.kernel,
        out_type=jax.ShapeDtypeStruct((2, _NPAD, _D), jnp.float32),
        mesh=_sc_mesh(),
        scratch_types=[
            pltpu.VMEM((_GCH,), jnp.int32),
            pltpu.VMEM((_GCH,), jnp.int32),
            pltpu.VMEM((_TAIL,), jnp.int32),
            pltpu.VMEM((_GCH, _D), jnp.float32),
            pltpu.VMEM_SHARED((_NPAD, _D), jnp.float32),
            pltpu.SemaphoreType.DMA,
            pltpu.SemaphoreType.DMA,
        ],
    )
    def k(r_hbm, ones_hbm, z_hbm, o_hbm, ib0, ib1, tib, ones_v, table,
          l0, l1):
        cid = lax.axis_index("c")
        sid = lax.axis_index("s")
        pltpu.sync_copy(ones_hbm, ones_v)
        pltpu.sync_copy(z_hbm.at[pl.ds(sid * _NPT, _NPT)],
                        table.at[pl.ds(sid * _NPT, _NPT)])
        plsc.subcore_barrier()
        base = cid * (_E // 2) + sid * _EPT
        ib = (ib0, ib1)
        lsem = (l0, l1)

        def idesc(c, p):
            return pltpu.make_async_copy(
                r_hbm.at[pl.ds(base + c * _GCH, _GCH)], ib[p], lsem[p])

        for p in (0, 1):
            idesc(p, p).start()

        @pl.loop(0, _NFULL, step=2)
        def _(i):
            for p in (0, 1):
                c = i + p
                idesc(c, p).wait()
                pltpu.sync_copy(ones_v, table.at[ib[p]], add=True)

                @pl.when(c + 2 < _NFULL)
                def _():
                    idesc(c + 2, p).start()

        tb = base + _NFULL * _GCH
        pltpu.sync_copy(r_hbm.at[pl.ds(tb, _TAIL)], tib)
        pltpu.sync_copy(ones_v.at[pl.ds(0, _TAIL)], table.at[tib], add=True)

        plsc.subcore_barrier()
        pltpu.sync_copy(table.at[pl.ds(sid * _NPT, _NPT)],
                        o_hbm.at[cid, pl.ds(sid * _NPT, _NPT)])

    return k(receivers, ones_chunk, zeros16)


def kernel(rnode_features, edge_features, senders, receivers, tau,
           emb_W1, emb_b1, emb_W2, emb_b2, emb_cW1, emb_cb1, emb_cW2,
           emb_cb2, e_W1, e_b1, e_W2, e_b2, e_cW1, e_cb1, e_cW2, e_cb2,
           n_W1, n_b1, n_W2, n_b2, n_cW1, n_cb1, n_cW2, n_cb2):
    v = rnode_features[0]
    ef = edge_features[0]

    def cond(cW1, cb1, cW2, cb2):
        p = _swish(tau @ cW1 + cb1) @ cW2 + cb2
        return 1.0 + p[:, :_D], p[:, _D:]

    zeros = jnp.zeros((_NPAD, _D), jnp.float32)
    ones_chunk = jnp.ones((_GCH, _D), jnp.float32)

    counts2 = _sc_counts(receivers, ones_chunk, zeros)
    counts = counts2[0, :, 0:1] + counts2[1, :, 0:1]
    inv_denom = 1.0 / jnp.maximum(counts, 1.0)

    _H = _E // 2
    s_halves = (senders[:_H], senders[_H:])
    r_halves = (receivers[:_H], receivers[_H:])

    emb_sc, emb_sh = cond(emb_cW1, emb_cb1, emb_cW2, emb_cb2)
    e_h = [
        _embed_call(ef[:_H], emb_W1, emb_b1[None], emb_W2, emb_b2[None],
                    emb_sc, emb_sh),
        _embed_call(ef[_H:], emb_W1, emb_b1[None], emb_W2, emb_b2[None],
                    emb_sc, emb_sh),
    ]

    w1splits = [(e_W1[st][:_D], e_W1[st][_D:2 * _D], e_W1[st][2 * _D:])
                for st in range(_STEPS)]
    vs, vr = _proj_call(v, w1splits[0][1], w1splits[0][2])
    for st in range(_STEPS):
        w1e = w1splits[st][0]
        e_sc, e_sh = cond(e_cW1[st], e_cb1[st], e_cW2[st], e_cb2[st])
        gsums = [_sc_gather(vs, vr, s_halves[h], r_halves[h], _H)
                 for h in range(2)]
        parts = []
        for h in range(2):
            e_h[h] = _edge_call(e_h[h], gsums[h], w1e, e_b1[st][None],
                                e_W2[st], e_b2[st][None], e_sc, e_sh)
            parts.append(_sc_scatter2(e_h[h], r_halves[h], zeros, _H))
        n1 = n_W1[st]
        w1v, w1a = n1[:_D], n1[_D:]
        n_sc, n_sh = cond(n_cW1[st], n_cb1[st], n_cW2[st], n_cb2[st])
        nst = (st + 1) % _STEPS
        v, vs, vr = _node_call(
            v, (parts[0][0], parts[0][1], parts[1][0], parts[1][1]),
            inv_denom, w1v, w1a, n_b1[st][None], n_W2[st], n_b2[st][None],
            n_sc, n_sh, w1splits[nst][1], w1splits[nst][2])

    return v[None]


# speedup vs baseline: 1.3449x; 1.0167x over previous
import functools

import jax
import jax.numpy as jnp
from jax import lax
from jax.experimental import pallas as pl
from jax.experimental.pallas import tpu as pltpu
from jax.experimental.pallas import tpu_sc as plsc

_N = 10000
_E = 320000
_D = 128
_STEPS = 4
_EB = 8000
_NB = 5000
_NTILES = 32
_EPT = _E // _NTILES
_NPAD = 10240
_NPT = _NPAD // 16


def _swish(x):
    return x * jax.nn.sigmoid(x)


def _norm_apply(q, sc, sh):
    mu = jnp.mean(q, axis=-1, keepdims=True)
    qc = q - mu
    var = jnp.mean(qc * qc, axis=-1, keepdims=True)
    return qc * lax.rsqrt(var + 1e-5) * sc + sh


def _full(shape):
    return pl.BlockSpec(shape, lambda i: tuple(0 for _ in shape))


def _rows(nb, d):
    return pl.BlockSpec((nb, d), lambda i: (i, 0))


def _embed_body(x_ref, w1_ref, b1_ref, w2_ref, b2_ref, sc_ref, sh_ref, o_ref):
    h = jnp.dot(x_ref[...], w1_ref[...], preferred_element_type=jnp.float32)
    h = h + b1_ref[...]
    h = _swish(h)
    q = jnp.dot(h, w2_ref[...], preferred_element_type=jnp.float32)
    q = q + b2_ref[...]
    o_ref[...] = _norm_apply(q, sc_ref[...], sh_ref[...])


def _embed_call(x, w1, b1, w2, b2, sc, sh):
    de = x.shape[-1]
    ne = x.shape[0]
    return pl.pallas_call(
        _embed_body,
        grid=(ne // _EB,),
        in_specs=[_rows(_EB, de), _full((de, _D)), _full((1, _D)),
                  _full((_D, _D)), _full((1, _D)), _full((1, _D)),
                  _full((1, _D))],
        out_specs=_rows(_EB, _D),
        out_shape=jax.ShapeDtypeStruct((ne, _D), jnp.float32),
    )(x, w1, b1, w2, b2, sc, sh)


def _proj_body(v_ref, ws_ref, wr_ref, vs_ref, vr_ref):
    v = v_ref[...]
    vs_ref[...] = jnp.dot(v, ws_ref[...], preferred_element_type=jnp.float32)
    vr_ref[...] = jnp.dot(v, wr_ref[...], preferred_element_type=jnp.float32)


def _proj_call(v, ws, wr):
    out = jax.ShapeDtypeStruct((_N, _D), jnp.float32)
    return pl.pallas_call(
        _proj_body,
        grid=(_N // _NB,),
        in_specs=[_rows(_NB, _D), _full((_D, _D)), _full((_D, _D))],
        out_specs=[_rows(_NB, _D), _rows(_NB, _D)],
        out_shape=[out, out],
    )(v, ws, wr)


def _edge_body(e_ref, g_ref, w1_ref, b1_ref, w2_ref, b2_ref,
               sc_ref, sh_ref, o_ref):
    e = e_ref[...]
    h = jnp.dot(e, w1_ref[...], preferred_element_type=jnp.float32)
    h = h + g_ref[...] + b1_ref[...]
    h = _swish(h)
    q = jnp.dot(h, w2_ref[...], preferred_element_type=jnp.float32)
    q = q + b2_ref[...]
    o_ref[...] = e + _norm_apply(q, sc_ref[...], sh_ref[...])


def _edge_call(e, gsum, w1, b1, w2, b2, sc, sh):
    eb = _rows(_EB, _D)
    ne = e.shape[0]
    return pl.pallas_call(
        _edge_body,
        grid=(ne // _EB,),
        in_specs=[eb, eb, _full((_D, _D)), _full((1, _D)),
                  _full((_D, _D)), _full((1, _D)), _full((1, _D)),
                  _full((1, _D))],
        out_specs=eb,
        out_shape=jax.ShapeDtypeStruct((ne, _D), jnp.float32),
    )(e, gsum, w1, b1, w2, b2, sc, sh)


def _node_body(v_ref, p0_ref, p1_ref, p2_ref, p3_ref, d_ref, w1v_ref,
               w1a_ref, b1_ref, w2_ref, b2_ref, sc_ref, sh_ref,
               ws_ref, wr_ref, o_ref, vs_ref, vr_ref):
    v = v_ref[...]
    agg = ((p0_ref[...] + p1_ref[...]) +
           (p2_ref[...] + p3_ref[...])) * d_ref[...]
    h = jnp.dot(v, w1v_ref[...], preferred_element_type=jnp.float32)
    h = h + jnp.dot(agg, w1a_ref[...], preferred_element_type=jnp.float32)
    h = h + b1_ref[...]
    h = _swish(h)
    q = jnp.dot(h, w2_ref[...], preferred_element_type=jnp.float32)
    q = q + b2_ref[...]
    vn = v + _norm_apply(q, sc_ref[...], sh_ref[...])
    o_ref[...] = vn
    vs_ref[...] = jnp.dot(vn, ws_ref[...], preferred_element_type=jnp.float32)
    vr_ref[...] = jnp.dot(vn, wr_ref[...], preferred_element_type=jnp.float32)


def _node_call(v, parts, inv_denom, w1v, w1a, b1, w2, b2, sc, sh, ws, wr):
    nb = _rows(_NB, _D)
    p0, p1, p2, p3 = parts
    out = jax.ShapeDtypeStruct((_N, _D), jnp.float32)
    return pl.pallas_call(
        _node_body,
        grid=(_N // _NB,),
        in_specs=[nb, nb, nb, nb, nb,
                  pl.BlockSpec((_NB, 1), lambda i: (i, 0)),
                  _full((_D, _D)), _full((_D, _D)), _full((1, _D)),
                  _full((_D, _D)), _full((1, _D)), _full((1, _D)),
                  _full((1, _D)), _full((_D, _D)), _full((_D, _D))],
        out_specs=[nb, nb, nb],
        out_shape=[out, out, out],
    )(v, p0, p1, p2, p3, inv_denom, w1v, w1a, b1, w2, b2, sc, sh, ws, wr)


def _sc_mesh():
    return plsc.VectorSubcoreMesh(core_axis_name="c", subcore_axis_name="s")


_GCH = 128
_NFULL = _EPT // _GCH
_TAIL = _EPT - _NFULL * _GCH


def _tile_plan(ept, slots=2, max_gch=128):
    for gch in range(max_gch, 0, -8):
        nfull = ept // gch
        tail = ept - nfull * gch
        if (nfull >= slots and nfull % slots == 0 and tail > 0
                and tail % 8 == 0):
            return gch, nfull, tail
    raise ValueError(ept)


def _sc_gather(vs, vr, senders, receivers, eh=_E):
    ept = eh // _NTILES
    nsl = 3
    gch, nfull, tail = _tile_plan(ept, nsl, max_gch=96)
    buf = pltpu.VMEM((gch, _D), jnp.float32)

    @functools.partial(
        pl.kernel,
        out_type=jax.ShapeDtypeStruct((eh, _D), jnp.float32),
        mesh=_sc_mesh(),
        scratch_types=(
            [pltpu.VMEM((ept,), jnp.int32)] * 2
            + [buf] * (3 * nsl)
            + [pltpu.VMEM((tail, _D), jnp.float32)] * 2
            + [pltpu.SemaphoreType.DMA] * (2 * nsl)
        ),
    )
    def k(vs_hbm, vr_hbm, s_hbm, r_hbm, o_hbm, *refs):
        idx_s, idx_r = refs[0], refs[1]
        bs = refs[2:2 + nsl]
        br = refs[2 + nsl:2 + 2 * nsl]
        wb = refs[2 + 2 * nsl:2 + 3 * nsl]
        tbs, tbr = refs[2 + 3 * nsl], refs[3 + 3 * nsl]
        gsem = refs[4 + 3 * nsl:4 + 4 * nsl]
        wsem = refs[4 + 4 * nsl:4 + 5 * nsl]
        wid = lax.axis_index("s") * 2 + lax.axis_index("c")
        base = wid * ept
        pltpu.sync_copy(s_hbm.at[pl.ds(base, ept)], idx_s)
        pltpu.sync_copy(r_hbm.at[pl.ds(base, ept)], idx_r)

        def gdescs(c, p):
            ds = pltpu.make_async_copy(
                vs_hbm.at[idx_s.at[pl.ds(c * gch, gch)]], bs[p], gsem[p])
            dr = pltpu.make_async_copy(
                vr_hbm.at[idx_r.at[pl.ds(c * gch, gch)]], br[p], gsem[p])
            return ds, dr

        def wdesc(c, p):
            return pltpu.make_async_copy(
                wb[p], o_hbm.at[pl.ds(base + c * gch, gch)], wsem[p])

        for p in range(nsl):
            ds, dr = gdescs(p, p)
            ds.start()
            dr.start()

        @pl.loop(0, nfull, step=nsl)
        def _(i):
            for p in range(nsl):
                c = i + p
                ds, dr = gdescs(c, p)
                ds.wait()
                dr.wait()

                @pl.when(c >= nsl)
                def _():
                    wdesc(c - nsl, p).wait()

                @pl.loop(0, gch)
                def _(r):
                    for j in range(_D // 16):
                        sl = pl.ds(j * 16, 16)
                        wb[p][r, sl] = bs[p][r, sl] + br[p][r, sl]

                wdesc(c, p).start()

                @pl.when(c + nsl < nfull)
                def _():
                    d2, r2 = gdescs(c + nsl, p)
                    d2.start()
                    r2.start()

        for p in range(nsl):
            wdesc(nfull - nsl + p, p).wait()

        tb = nfull * gch
        pltpu.sync_copy(vs_hbm.at[idx_s.at[pl.ds(tb, tail)]], tbs)
        pltpu.sync_copy(vr_hbm.at[idx_r.at[pl.ds(tb, tail)]], tbr)

        @pl.loop(0, tail)
        def _(r):
            for j in range(_D // 16):
                sl = pl.ds(j * 16, 16)
                tbs[r, sl] = tbs[r, sl] + tbr[r, sl]

        pltpu.sync_copy(tbs, o_hbm.at[pl.ds(base + tb, tail)])

    return k(vs, vr, senders, receivers)


def _sc_scatter2(e, receivers, zeros, eh=_E):
    ept = eh // _NTILES
    gch, nfull, tail = _tile_plan(ept)

    @functools.partial(
        pl.kernel,
        out_type=jax.ShapeDtypeStruct((2, _NPAD, _D), jnp.float32),
        mesh=_sc_mesh(),
        scratch_types=[
            pltpu.VMEM((gch,), jnp.int32),
            pltpu.VMEM((gch,), jnp.int32),
            pltpu.VMEM((gch, _D), jnp.float32),
            pltpu.VMEM((gch, _D), jnp.float32),
            pltpu.VMEM((tail,), jnp.int32),
            pltpu.VMEM((tail, _D), jnp.float32),
            pltpu.VMEM_SHARED((_NPAD, _D), jnp.float32),
            pltpu.SemaphoreType.DMA,
            pltpu.SemaphoreType.DMA,
        ],
    )
    def k(e_hbm, r_hbm, z_hbm, o_hbm, ib0, ib1, eb0, eb1, tib, teb, table,
          l0, l1):
        cid = lax.axis_index("c")
        sid = lax.axis_index("s")
        pltpu.sync_copy(z_hbm.at[pl.ds(sid * _NPT, _NPT)],
                        table.at[pl.ds(sid * _NPT, _NPT)])
        plsc.subcore_barrier()
        base = cid * (eh // 2) + sid * ept
        ib = (ib0, ib1)
        eb = (eb0, eb1)
        lsem = (l0, l1)

        def ldescs(c, p):
            b = base + c * gch
            di = pltpu.make_async_copy(r_hbm.at[pl.ds(b, gch)], ib[p],
                                       lsem[p])
            de = pltpu.make_async_copy(e_hbm.at[pl.ds(b, gch)], eb[p],
                                       lsem[p])
            return di, de

        for p in (0, 1):
            di, de = ldescs(p, p)
            di.start()
            de.start()

        @pl.loop(0, nfull, step=2)
        def _(i):
            for p in (0, 1):
                c = i + p
                di, de = ldescs(c, p)
                di.wait()
                de.wait()
                pltpu.sync_copy(eb[p], table.at[ib[p]], add=True)

                @pl.when(c + 2 < nfull)
                def _():
                    d2, e2 = ldescs(c + 2, p)
                    d2.start()
                    e2.start()

        tb = base + nfull * gch
        pltpu.sync_copy(r_hbm.at[pl.ds(tb, tail)], tib)
        pltpu.sync_copy(e_hbm.at[pl.ds(tb, tail)], teb)
        pltpu.sync_copy(teb, table.at[tib], add=True)

        plsc.subcore_barrier()
        pltpu.sync_copy(table.at[pl.ds(sid * _NPT, _NPT)],
                        o_hbm.at[cid, pl.ds(sid * _NPT, _NPT)])

    return k(e, receivers, zeros)


def _sc_counts(receivers, ones_chunk, zeros16):
    @functools.partial(
        pl.kernel,
        out_type=jax.ShapeDtypeStruct((2, _NPAD, _D), jnp.float32),
        mesh=_sc_mesh(),
        scratch_types=[
            pltpu.VMEM((_GCH,), jnp.int32),
            pltpu.VMEM((_GCH,), jnp.int32),
            pltpu.VMEM((_TAIL,), jnp.int32),
            pltpu.VMEM((_GCH, _D), jnp.float32),
            pltpu.VMEM_SHARED((_NPAD, _D), jnp.float32),
            pltpu.SemaphoreType.DMA,
            pltpu.SemaphoreType.DMA,
        ],
    )
    def k(r_hbm, ones_hbm, z_hbm, o_hbm, ib0, ib1, tib, ones_v, table,
          l0, l1):
        cid = lax.axis_index("c")
        sid = lax.axis_index("s")
        pltpu.sync_copy(ones_hbm, ones_v)
        pltpu.sync_copy(z_hbm.at[pl.ds(sid * _NPT, _NPT)],
                        table.at[pl.ds(sid * _NPT, _NPT)])
        plsc.subcore_barrier()
        base = cid * (_E // 2) + sid * _EPT
        ib = (ib0, ib1)
        lsem = (l0, l1)

        def idesc(c, p):
            return pltpu.make_async_copy(
                r_hbm.at[pl.ds(base + c * _GCH, _GCH)], ib[p], lsem[p])

        for p in (0, 1):
            idesc(p, p).start()

        @pl.loop(0, _NFULL, step=2)
        def _(i):
            for p in (0, 1):
                c = i + p
                idesc(c, p).wait()
                pltpu.sync_copy(ones_v, table.at[ib[p]], add=True)

                @pl.when(c + 2 < _NFULL)
                def _():
                    idesc(c + 2, p).start()

        tb = base + _NFULL * _GCH
        pltpu.sync_copy(r_hbm.at[pl.ds(tb, _TAIL)], tib)
        pltpu.sync_copy(ones_v.at[pl.ds(0, _TAIL)], table.at[tib], add=True)

        plsc.subcore_barrier()
        pltpu.sync_copy(table.at[pl.ds(sid * _NPT, _NPT)],
                        o_hbm.at[cid, pl.ds(sid * _NPT, _NPT)])

    return k(receivers, ones_chunk, zeros16)


def kernel(rnode_features, edge_features, senders, receivers, tau,
           emb_W1, emb_b1, emb_W2, emb_b2, emb_cW1, emb_cb1, emb_cW2,
           emb_cb2, e_W1, e_b1, e_W2, e_b2, e_cW1, e_cb1, e_cW2, e_cb2,
           n_W1, n_b1, n_W2, n_b2, n_cW1, n_cb1, n_cW2, n_cb2):
    v = rnode_features[0]
    ef = edge_features[0]

    def cond(cW1, cb1, cW2, cb2):
        p = _swish(tau @ cW1 + cb1) @ cW2 + cb2
        return 1.0 + p[:, :_D], p[:, _D:]

    zeros = jnp.zeros((_NPAD, _D), jnp.float32)
    ones_chunk = jnp.ones((_GCH, _D), jnp.float32)

    counts2 = _sc_counts(receivers, ones_chunk, zeros)
    counts = counts2[0, :, 0:1] + counts2[1, :, 0:1]
    inv_denom = 1.0 / jnp.maximum(counts, 1.0)

    _H = _E // 2
    s_halves = (senders[:_H], senders[_H:])
    r_halves = (receivers[:_H], receivers[_H:])

    emb_sc, emb_sh = cond(emb_cW1, emb_cb1, emb_cW2, emb_cb2)
    e_h = [
        _embed_call(ef[:_H], emb_W1, emb_b1[None], emb_W2, emb_b2[None],
                    emb_sc, emb_sh),
        _embed_call(ef[_H:], emb_W1, emb_b1[None], emb_W2, emb_b2[None],
                    emb_sc, emb_sh),
    ]

    w1splits = [(e_W1[st][:_D], e_W1[st][_D:2 * _D], e_W1[st][2 * _D:])
                for st in range(_STEPS)]
    vs, vr = _proj_call(v, w1splits[0][1], w1splits[0][2])
    for st in range(_STEPS):
        w1e = w1splits[st][0]
        e_sc, e_sh = cond(e_cW1[st], e_cb1[st], e_cW2[st], e_cb2[st])
        gsums = [_sc_gather(vs, vr, s_halves[h], r_halves[h], _H)
                 for h in range(2)]
        parts = []
        for h in range(2):
            e_h[h] = _edge_call(e_h[h], gsums[h], w1e, e_b1[st][None],
                                e_W2[st], e_b2[st][None], e_sc, e_sh)
            parts.append(_sc_scatter2(e_h[h], r_halves[h], zeros, _H))
        n1 = n_W1[st]
        w1v, w1a = n1[:_D], n1[_D:]
        n_sc, n_sh = cond(n_cW1[st], n_cb1[st], n_cW2[st], n_cb2[st])
        nst = (st + 1) % _STEPS
        v, vs, vr = _node_call(
            v, (parts[0][0], parts[0][1], parts[1][0], parts[1][1]),
            inv_denom, w1v, w1a, n_b1[st][None], n_W2[st], n_b2[st][None],
            n_sc, n_sh, w1splits[nst][1], w1splits[nst][2])

    return v[None]
